# baseline (speedup 1.0000x reference)
import jax
import jax.numpy as jnp
from jax import lax
from jax.experimental import pallas as pl
from jax.experimental.pallas import tpu as pltpu
from jax.experimental.pallas import tpu_sc as plsc

N_TOKENS = 16384
NGRAMS = 12
VAL_DIM = 32
KEY_DIM = 64
NUM_LM = 32
TOKEN_DIM = 256
KEY_PAD = 1024

NC = 2
NS = 16
NW = NC * NS
TPW = N_TOKENS // NW
CHUNK = 32
N_CHUNKS = TPW // CHUNK
ROWS_PER_CHUNK = CHUNK * NGRAMS
BURSTS = ROWS_PER_CHUNK // 128
Q_ROWS = TPW * NGRAMS // 128


def _sc_body(tab4, q_idx, mrep, val_out, q_v, m_v, rows_v, val_cv, sem):
    wid = lax.axis_index("s") * NC + lax.axis_index("c")
    pltpu.sync_copy(q_idx.at[wid], q_v)

    def chunk_body(c, carry):
        pltpu.sync_copy(mrep.at[wid, c], m_v)
        copies = [
            pltpu.async_copy(tab4.at[q_v.at[c * BURSTS + r]],
                             rows_v.at[pl.ds(r * 128, 128)], sem)
            for r in range(BURSTS)
        ]
        for cp in copies:
            cp.wait()

        def tok_body(t, carry2):
            def mvec(r):
                return m_v[r // 8, pl.ds((r % 8) * 16, 16)]
            mvecs = [mvec(t * NGRAMS + j) for j in range(NGRAMS)]
            for h in range(VAL_DIM // 16):
                acc = jnp.zeros((16,), jnp.float32)
                for j in range(NGRAMS):
                    r = t * NGRAMS + j
                    for k in range(4):
                        v = rows_v[r, pl.ds(k * VAL_DIM + h * 16, 16)]
                        acc = acc + jnp.where(mvecs[j] == k, v, 0.0)
                val_cv[t, pl.ds(h * 16, 16)] = acc * (1.0 / NGRAMS)
            return carry2

        lax.fori_loop(0, CHUNK, tok_body, 0)
        pltpu.sync_copy(val_cv,
                        val_out.at[pl.ds(wid * TPW + c * CHUNK, CHUNK)])
        return carry

    lax.fori_loop(0, N_CHUNKS, chunk_body, 0)


@jax.jit
def _sc_bag(tab4, q_idx, mrep):
    mesh = plsc.VectorSubcoreMesh(core_axis_name="c", subcore_axis_name="s")
    f = pl.kernel(
        _sc_body,
        out_type=jax.ShapeDtypeStruct((N_TOKENS, 128), jnp.float32),
        mesh=mesh,
        compiler_params=pltpu.CompilerParams(use_tc_tiling_on_sc=True),
        scratch_types=[
            pltpu.VMEM((Q_ROWS, 128), jnp.int32),
            pltpu.VMEM((ROWS_PER_CHUNK * 16 // 128, 128), jnp.int32),
            pltpu.VMEM((ROWS_PER_CHUNK, 128), jnp.float32),
            pltpu.VMEM((CHUNK, 128), jnp.float32),
            pltpu.SemaphoreType.DMA,
        ],
    )
    return f(tab4, q_idx, mrep)


TB = 512


def _tc_a_body(desc_ref, kidx_ref, lmi_ref, dW_ref, db_ref, ktp_ref, wk_ref,
               wd_ref, tb_ref, lmt_ref, out_ref):
    descp = jnp.dot(desc_ref[...], dW_ref[...],
                    preferred_element_type=jnp.float32) + db_ref[...]
    acc = jnp.dot(descp, wd_ref[...], preferred_element_type=jnp.float32)
    kidx = kidx_ref[0]
    ohk = (kidx == lax.broadcasted_iota(jnp.int32, (TB, KEY_PAD), 1))
    key_e = jnp.dot(ohk.astype(jnp.float32), ktp_ref[...],
                    preferred_element_type=jnp.float32)
    acc = acc + jnp.dot(key_e, wk_ref[...], preferred_element_type=jnp.float32)
    lmi = lmi_ref[0]
    ohl = (lmi == lax.broadcasted_iota(jnp.int32, (TB, NUM_LM), 1))
    acc = acc + jnp.dot(ohl.astype(jnp.float32), lmt_ref[...],
                        preferred_element_type=jnp.float32)
    out_ref[...] = acc + tb_ref[...]


@jax.jit
def _tc_a(desc_emb, kidx3, lmi3, desc_W, db2, ktp_w, wk, wd, tb2, lm_table):
    n = desc_emb.shape[0]
    return pl.pallas_call(
        _tc_a_body,
        grid=(n // TB,),
        in_specs=[
            pl.BlockSpec((TB, desc_emb.shape[1]), lambda i: (i, 0)),
            pl.BlockSpec((1, TB, 1), lambda i: (i, 0, 0)),
            pl.BlockSpec((1, TB, 1), lambda i: (i, 0, 0)),
            pl.BlockSpec(desc_W.shape, lambda i: (0, 0)),
            pl.BlockSpec(db2.shape, lambda i: (0, 0)),
            pl.BlockSpec(ktp_w.shape, lambda i: (0, 0)),
            pl.BlockSpec(wk.shape, lambda i: (0, 0)),
            pl.BlockSpec(wd.shape, lambda i: (0, 0)),
            pl.BlockSpec(tb2.shape, lambda i: (0, 0)),
            pl.BlockSpec(lm_table.shape, lambda i: (0, 0)),
        ],
        out_specs=pl.BlockSpec((TB, TOKEN_DIM), lambda i: (i, 0)),
        out_shape=jax.ShapeDtypeStruct((n, TOKEN_DIM), jnp.float32),
    )(desc_emb, kidx3, lmi3, desc_W, db2, ktp_w, wk, wd, tb2, lm_table)


def _tc_b_body(part_ref, val_ref, wv_ref, out_ref):
    val = val_ref[...][:, :VAL_DIM]
    contrib = jnp.dot(val, wv_ref[...], preferred_element_type=jnp.float32)
    out_ref[...] = part_ref[...] + contrib


@jax.jit
def _tc_b(partial, val_pad, wv):
    n = partial.shape[0]
    return pl.pallas_call(
        _tc_b_body,
        grid=(n // TB,),
        in_specs=[
            pl.BlockSpec((TB, TOKEN_DIM), lambda i: (i, 0)),
            pl.BlockSpec((TB, 128), lambda i: (i, 0)),
            pl.BlockSpec(wv.shape, lambda i: (0, 0)),
        ],
        out_specs=pl.BlockSpec((TB, TOKEN_DIM), lambda i: (i, 0)),
        out_shape=jax.ShapeDtypeStruct((n, TOKEN_DIM), jnp.float32),
    )(partial, val_pad, wv)


def kernel(key_idx, ngram_idx, landmark_idx, desc_emb, key_table, ngram_table,
           desc_W, desc_b, tag_W, tag_b, lm_table):
    n, g = ngram_idx.shape
    ngi = ngram_idx.astype(jnp.int32)
    tab4 = ngram_table.reshape(ngram_table.shape[0] // 4, 128)
    q_idx = (ngi // 4).reshape(NW, Q_ROWS, 128)
    mrep = jnp.broadcast_to(
        (ngi % 4).reshape(NW, N_CHUNKS, ROWS_PER_CHUNK, 1),
        (NW, N_CHUNKS, ROWS_PER_CHUNK, 16)
    ).reshape(NW, N_CHUNKS, ROWS_PER_CHUNK * 16 // 128, 128)
    val_pad = _sc_bag(tab4, q_idx, mrep)

    kidx3 = key_idx.astype(jnp.int32).reshape(n // TB, TB, 1)
    lmi3 = landmark_idx.astype(jnp.int32).reshape(n // TB, TB, 1)
    db2 = desc_b.reshape(1, -1)
    tb2 = tag_b.reshape(1, -1)
    ktp_w = jnp.zeros((KEY_PAD, KEY_DIM), jnp.float32).at[:key_table.shape[0]].set(key_table)
    wk = tag_W[:KEY_DIM]
    wv = tag_W[KEY_DIM:KEY_DIM + VAL_DIM]
    wd = tag_W[KEY_DIM + VAL_DIM:]
    partial = _tc_a(desc_emb, kidx3, lmi3, desc_W, db2, ktp_w, wk, wd, tb2,
                    lm_table)
    return _tc_b(partial, val_pad, wv)


# speedup vs baseline: 1.0818x; 1.0818x over previous
import jax
import jax.numpy as jnp
from jax import lax
from jax.experimental import pallas as pl
from jax.experimental.pallas import tpu as pltpu
from jax.experimental.pallas import tpu_sc as plsc

N_TOKENS = 16384
NGRAMS = 12
VAL_DIM = 32
KEY_DIM = 64
NUM_LM = 32
TOKEN_DIM = 256
KEY_PAD = 1024
NGRAM_V = 1000000

NC = 2
NS = 16
NW = NC * NS
TPW = N_TOKENS // NW
CHUNK = 32
N_CHUNKS = TPW // CHUNK
ROWS_PER_CHUNK = CHUNK * NGRAMS
BURSTS = ROWS_PER_CHUNK // 128
Q_ROWS = TPW * NGRAMS // 128

PACK_W = 2048
PACK_BLKS = (NGRAM_V + PACK_W - 1) // PACK_W
F_ROWS = PACK_BLKS * (PACK_W // 4)


def _pack_body(x_ref, o_ref):
    ys = []
    for k in range(4):
        ys.append(lax.transpose(x_ref[:, k * 512:(k + 1) * 512], (1, 0)))
    o_ref[...] = jnp.concatenate(ys, axis=1)


@jax.jit
def _tc_pack(tabT):
    return pl.pallas_call(
        _pack_body,
        grid=(PACK_BLKS,),
        in_specs=[pl.BlockSpec((VAL_DIM, PACK_W), lambda i: (0, i))],
        out_specs=pl.BlockSpec((512, 128), lambda i: (i, 0)),
        out_shape=jax.ShapeDtypeStruct((F_ROWS, 128), jnp.float32),
    )(tabT)


def _sc_body(tabF, q_idx, mrep, val_out, q_v, m_v, rows_v, val_cv, sem):
    wid = lax.axis_index("s") * NC + lax.axis_index("c")
    pltpu.sync_copy(q_idx.at[wid], q_v)

    def chunk_body(c, carry):
        pltpu.sync_copy(mrep.at[wid, c], m_v)
        copies = [
            pltpu.async_copy(tabF.at[q_v.at[c * BURSTS + r]],
                             rows_v.at[pl.ds(r * 128, 128)], sem)
            for r in range(BURSTS)
        ]
        for cp in copies:
            cp.wait()

        def tok_body(t, carry2):
            sel = []
            for j in range(NGRAMS):
                f = t * NGRAMS + j
                mvec = m_v[f // 8, pl.ds((f % 8) * 16, 16)]
                sel.append((mvec < 2, lax.rem(mvec, 2) == 0))
            for h in range(VAL_DIM // 16):
                acc = jnp.zeros((16,), jnp.float32)
                for j in range(NGRAMS):
                    r = t * NGRAMS + j
                    lo01, even = sel[j]
                    v0 = rows_v[r, pl.ds(0 * VAL_DIM + h * 16, 16)]
                    v1 = rows_v[r, pl.ds(1 * VAL_DIM + h * 16, 16)]
                    v2 = rows_v[r, pl.ds(2 * VAL_DIM + h * 16, 16)]
                    v3 = rows_v[r, pl.ds(3 * VAL_DIM + h * 16, 16)]
                    lo = jnp.where(even, v0, v1)
                    hi = jnp.where(even, v2, v3)
                    acc = acc + jnp.where(lo01, lo, hi)
                val_cv[t, pl.ds(h * 16, 16)] = acc * (1.0 / NGRAMS)
            return carry2

        lax.fori_loop(0, CHUNK, tok_body, 0)
        pltpu.sync_copy(val_cv,
                        val_out.at[pl.ds(wid * TPW + c * CHUNK, CHUNK)])
        return carry

    lax.fori_loop(0, N_CHUNKS, chunk_body, 0)


@jax.jit
def _sc_bag(tabF, q_idx, mrep):
    mesh = plsc.VectorSubcoreMesh(core_axis_name="c", subcore_axis_name="s")
    f = pl.kernel(
        _sc_body,
        out_type=jax.ShapeDtypeStruct((N_TOKENS, 128), jnp.float32),
        mesh=mesh,
        scratch_types=[
            pltpu.VMEM((Q_ROWS, 128), jnp.int32),
            pltpu.VMEM((ROWS_PER_CHUNK * 16 // 128, 128), jnp.int32),
            pltpu.VMEM((ROWS_PER_CHUNK, 128), jnp.float32),
            pltpu.VMEM((CHUNK, 128), jnp.float32),
            pltpu.SemaphoreType.DMA,
        ],
    )
    return f(tabF, q_idx, mrep)


TB = 512


def _tc_a_body(desc_ref, kidx_ref, lmi_ref, dW_ref, db_ref, ktp_ref, wk_ref,
               wd_ref, tb_ref, lmt_ref, out_ref):
    descp = jnp.dot(desc_ref[...], dW_ref[...],
                    preferred_element_type=jnp.float32) + db_ref[...]
    acc = jnp.dot(descp, wd_ref[...], preferred_element_type=jnp.float32)
    kidx = kidx_ref[0]
    ohk = (kidx == lax.broadcasted_iota(jnp.int32, (TB, KEY_PAD), 1))
    key_e = jnp.dot(ohk.astype(jnp.float32), ktp_ref[...],
                    preferred_element_type=jnp.float32)
    acc = acc + jnp.dot(key_e, wk_ref[...], preferred_element_type=jnp.float32)
    lmi = lmi_ref[0]
    ohl = (lmi == lax.broadcasted_iota(jnp.int32, (TB, NUM_LM), 1))
    acc = acc + jnp.dot(ohl.astype(jnp.float32), lmt_ref[...],
                        preferred_element_type=jnp.float32)
    out_ref[...] = acc + tb_ref[...]


@jax.jit
def _tc_a(desc_emb, kidx3, lmi3, desc_W, db2, ktp_w, wk, wd, tb2, lm_table):
    n = desc_emb.shape[0]
    return pl.pallas_call(
        _tc_a_body,
        grid=(n // TB,),
        in_specs=[
            pl.BlockSpec((TB, desc_emb.shape[1]), lambda i: (i, 0)),
            pl.BlockSpec((1, TB, 1), lambda i: (i, 0, 0)),
            pl.BlockSpec((1, TB, 1), lambda i: (i, 0, 0)),
            pl.BlockSpec(desc_W.shape, lambda i: (0, 0)),
            pl.BlockSpec(db2.shape, lambda i: (0, 0)),
            pl.BlockSpec(ktp_w.shape, lambda i: (0, 0)),
            pl.BlockSpec(wk.shape, lambda i: (0, 0)),
            pl.BlockSpec(wd.shape, lambda i: (0, 0)),
            pl.BlockSpec(tb2.shape, lambda i: (0, 0)),
            pl.BlockSpec(lm_table.shape, lambda i: (0, 0)),
        ],
        out_specs=pl.BlockSpec((TB, TOKEN_DIM), lambda i: (i, 0)),
        out_shape=jax.ShapeDtypeStruct((n, TOKEN_DIM), jnp.float32),
    )(desc_emb, kidx3, lmi3, desc_W, db2, ktp_w, wk, wd, tb2, lm_table)


def _tc_b_body(part_ref, val_ref, wv_ref, out_ref):
    val = val_ref[...][:, :VAL_DIM]
    contrib = jnp.dot(val, wv_ref[...], preferred_element_type=jnp.float32)
    out_ref[...] = part_ref[...] + contrib


@jax.jit
def _tc_b(partial, val_pad, wv):
    n = partial.shape[0]
    return pl.pallas_call(
        _tc_b_body,
        grid=(n // TB,),
        in_specs=[
            pl.BlockSpec((TB, TOKEN_DIM), lambda i: (i, 0)),
            pl.BlockSpec((TB, 128), lambda i: (i, 0)),
            pl.BlockSpec(wv.shape, lambda i: (0, 0)),
        ],
        out_specs=pl.BlockSpec((TB, TOKEN_DIM), lambda i: (i, 0)),
        out_shape=jax.ShapeDtypeStruct((n, TOKEN_DIM), jnp.float32),
    )(partial, val_pad, wv)


def kernel(key_idx, ngram_idx, landmark_idx, desc_emb, key_table, ngram_table,
           desc_W, desc_b, tag_W, tag_b, lm_table):
    n, g = ngram_idx.shape
    ngi = ngram_idx.astype(jnp.int32)
    tabF = _tc_pack(ngram_table.T)
    q = (ngi >> 11) * 512 + (ngi & 511)
    q_idx = q.reshape(NW, Q_ROWS, 128)
    m = (ngi >> 9) & 3
    mrep = jnp.broadcast_to(
        m.reshape(NW, N_CHUNKS, ROWS_PER_CHUNK, 1),
        (NW, N_CHUNKS, ROWS_PER_CHUNK, 16)
    ).reshape(NW, N_CHUNKS, ROWS_PER_CHUNK * 16 // 128, 128)
    val_pad = _sc_bag(tabF, q_idx, mrep)

    kidx3 = key_idx.astype(jnp.int32).reshape(n // TB, TB, 1)
    lmi3 = landmark_idx.astype(jnp.int32).reshape(n // TB, TB, 1)
    db2 = desc_b.reshape(1, -1)
    tb2 = tag_b.reshape(1, -1)
    ktp_w = jnp.zeros((KEY_PAD, KEY_DIM), jnp.float32).at[:key_table.shape[0]].set(key_table)
    wk = tag_W[:KEY_DIM]
    wv = tag_W[KEY_DIM:KEY_DIM + VAL_DIM]
    wd = tag_W[KEY_DIM + VAL_DIM:]
    partial = _tc_a(desc_emb, kidx3, lmi3, desc_W, db2, ktp_w, wk, wd, tb2,
                    lm_table)
    return _tc_b(partial, val_pad, wv)


# speedup vs baseline: 1.4386x; 1.3298x over previous
import jax
import jax.numpy as jnp
from jax import lax
from jax.experimental import pallas as pl
from jax.experimental.pallas import tpu as pltpu
from jax.experimental.pallas import tpu_sc as plsc

N_TOKENS = 16384
NGRAMS = 12
VAL_DIM = 32
KEY_DIM = 64
NUM_LM = 32
TOKEN_DIM = 256
KEY_PAD = 1024
NGRAM_V = 1000000

NC = 2
NS = 16
NW = NC * NS
TPW = N_TOKENS // NW
CHUNK = 32
N_CHUNKS = TPW // CHUNK
ROWS_PER_CHUNK = CHUNK * NGRAMS
BURSTS = ROWS_PER_CHUNK // 128
Q_ROWS = TPW * NGRAMS // 128

PACK_W = 8192
PACK_BLKS = (NGRAM_V + PACK_W - 1) // PACK_W
F_ROWS = PACK_BLKS * (PACK_W // 4)


def _pack_body(x_ref, o_ref):
    eye = (lax.broadcasted_iota(jnp.int32, (VAL_DIM, VAL_DIM), 0)
           == lax.broadcasted_iota(jnp.int32, (VAL_DIM, VAL_DIM), 1)
           ).astype(jnp.float32)
    ys = []
    for k in range(4):
        ys.append(lax.dot_general(x_ref[:, k * (PACK_W // 4):(k + 1) * (PACK_W // 4)], eye,
                                  (((0,), (0,)), ((), ())),
                                  preferred_element_type=jnp.float32))
    o_ref[...] = jnp.concatenate(ys, axis=1)


@jax.jit
def _tc_pack(tabT):
    return pl.pallas_call(
        _pack_body,
        grid=(PACK_BLKS,),
        in_specs=[pl.BlockSpec((VAL_DIM, PACK_W), lambda i: (0, i))],
        out_specs=pl.BlockSpec((PACK_W // 4, 128), lambda i: (i, 0)),
        out_shape=jax.ShapeDtypeStruct((F_ROWS, 128), jnp.float32),
    )(tabT)


def _sc_body(tabF, q_idx, mrep, val_out, q_v, m_v, rows_v, val_cv, sem):
    wid = lax.axis_index("s") * NC + lax.axis_index("c")
    pltpu.sync_copy(q_idx.at[wid], q_v)

    def chunk_body(c, carry):
        pltpu.sync_copy(mrep.at[wid, c], m_v)
        copies = [
            pltpu.async_copy(tabF.at[q_v.at[c * BURSTS + r]],
                             rows_v.at[pl.ds(r * 128, 128)], sem)
            for r in range(BURSTS)
        ]
        for cp in copies:
            cp.wait()

        def tok_body(t, carry2):
            sel = []
            for j in range(NGRAMS):
                f = t * NGRAMS + j
                mvec = m_v[f // 8, pl.ds((f % 8) * 16, 16)]
                sel.append((mvec < 2, lax.rem(mvec, 2) == 0))
            for h in range(VAL_DIM // 16):
                acc = jnp.zeros((16,), jnp.float32)
                for j in range(NGRAMS):
                    r = t * NGRAMS + j
                    lo01, even = sel[j]
                    v0 = rows_v[r, pl.ds(0 * VAL_DIM + h * 16, 16)]
                    v1 = rows_v[r, pl.ds(1 * VAL_DIM + h * 16, 16)]
                    v2 = rows_v[r, pl.ds(2 * VAL_DIM + h * 16, 16)]
                    v3 = rows_v[r, pl.ds(3 * VAL_DIM + h * 16, 16)]
                    lo = jnp.where(even, v0, v1)
                    hi = jnp.where(even, v2, v3)
                    acc = acc + jnp.where(lo01, lo, hi)
                val_cv[t, pl.ds(h * 16, 16)] = acc * (1.0 / NGRAMS)
            return carry2

        lax.fori_loop(0, CHUNK, tok_body, 0)
        pltpu.sync_copy(val_cv,
                        val_out.at[pl.ds(wid * TPW + c * CHUNK, CHUNK)])
        return carry

    lax.fori_loop(0, N_CHUNKS, chunk_body, 0)


@jax.jit
def _sc_bag(tabF, q_idx, mrep):
    mesh = plsc.VectorSubcoreMesh(core_axis_name="c", subcore_axis_name="s")
    f = pl.kernel(
        _sc_body,
        out_type=jax.ShapeDtypeStruct((N_TOKENS, 128), jnp.float32),
        mesh=mesh,
        scratch_types=[
            pltpu.VMEM((Q_ROWS, 128), jnp.int32),
            pltpu.VMEM((ROWS_PER_CHUNK * 16 // 128, 128), jnp.int32),
            pltpu.VMEM((ROWS_PER_CHUNK, 128), jnp.float32),
            pltpu.VMEM((CHUNK, 128), jnp.float32),
            pltpu.SemaphoreType.DMA,
        ],
    )
    return f(tabF, q_idx, mrep)


TB = 512


def _tc_a_body(desc_ref, kidx_ref, lmi_ref, dW_ref, db_ref, ktp_ref, wk_ref,
               wd_ref, tb_ref, lmt_ref, out_ref):
    descp = jnp.dot(desc_ref[...], dW_ref[...],
                    preferred_element_type=jnp.float32) + db_ref[...]
    acc = jnp.dot(descp, wd_ref[...], preferred_element_type=jnp.float32)
    kidx = kidx_ref[0]
    ohk = (kidx == lax.broadcasted_iota(jnp.int32, (TB, KEY_PAD), 1))
    key_e = jnp.dot(ohk.astype(jnp.float32), ktp_ref[...],
                    preferred_element_type=jnp.float32)
    acc = acc + jnp.dot(key_e, wk_ref[...], preferred_element_type=jnp.float32)
    lmi = lmi_ref[0]
    ohl = (lmi == lax.broadcasted_iota(jnp.int32, (TB, NUM_LM), 1))
    acc = acc + jnp.dot(ohl.astype(jnp.float32), lmt_ref[...],
                        preferred_element_type=jnp.float32)
    out_ref[...] = acc + tb_ref[...]


@jax.jit
def _tc_a(desc_emb, kidx3, lmi3, desc_W, db2, ktp_w, wk, wd, tb2, lm_table):
    n = desc_emb.shape[0]
    return pl.pallas_call(
        _tc_a_body,
        grid=(n // TB,),
        in_specs=[
            pl.BlockSpec((TB, desc_emb.shape[1]), lambda i: (i, 0)),
            pl.BlockSpec((1, TB, 1), lambda i: (i, 0, 0)),
            pl.BlockSpec((1, TB, 1), lambda i: (i, 0, 0)),
            pl.BlockSpec(desc_W.shape, lambda i: (0, 0)),
            pl.BlockSpec(db2.shape, lambda i: (0, 0)),
            pl.BlockSpec(ktp_w.shape, lambda i: (0, 0)),
            pl.BlockSpec(wk.shape, lambda i: (0, 0)),
            pl.BlockSpec(wd.shape, lambda i: (0, 0)),
            pl.BlockSpec(tb2.shape, lambda i: (0, 0)),
            pl.BlockSpec(lm_table.shape, lambda i: (0, 0)),
        ],
        out_specs=pl.BlockSpec((TB, TOKEN_DIM), lambda i: (i, 0)),
        out_shape=jax.ShapeDtypeStruct((n, TOKEN_DIM), jnp.float32),
    )(desc_emb, kidx3, lmi3, desc_W, db2, ktp_w, wk, wd, tb2, lm_table)


def _tc_b_body(part_ref, val_ref, wv_ref, out_ref):
    val = val_ref[...][:, :VAL_DIM]
    contrib = jnp.dot(val, wv_ref[...], preferred_element_type=jnp.float32)
    out_ref[...] = part_ref[...] + contrib


@jax.jit
def _tc_b(partial, val_pad, wv):
    n = partial.shape[0]
    return pl.pallas_call(
        _tc_b_body,
        grid=(n // TB,),
        in_specs=[
            pl.BlockSpec((TB, TOKEN_DIM), lambda i: (i, 0)),
            pl.BlockSpec((TB, 128), lambda i: (i, 0)),
            pl.BlockSpec(wv.shape, lambda i: (0, 0)),
        ],
        out_specs=pl.BlockSpec((TB, TOKEN_DIM), lambda i: (i, 0)),
        out_shape=jax.ShapeDtypeStruct((n, TOKEN_DIM), jnp.float32),
    )(partial, val_pad, wv)


def kernel(key_idx, ngram_idx, landmark_idx, desc_emb, key_table, ngram_table,
           desc_W, desc_b, tag_W, tag_b, lm_table):
    n, g = ngram_idx.shape
    ngi = ngram_idx.astype(jnp.int32)
    tabF = _tc_pack(ngram_table.T)
    pw4 = PACK_W // 4
    q = (ngi // PACK_W) * pw4 + (ngi % pw4)
    q_idx = q.reshape(NW, Q_ROWS, 128)
    m = (ngi // pw4) % 4
    mrep = jnp.broadcast_to(
        m.reshape(NW, N_CHUNKS, ROWS_PER_CHUNK, 1),
        (NW, N_CHUNKS, ROWS_PER_CHUNK, 16)
    ).reshape(NW, N_CHUNKS, ROWS_PER_CHUNK * 16 // 128, 128)
    val_pad = _sc_bag(tabF, q_idx, mrep)

    kidx3 = key_idx.astype(jnp.int32).reshape(n // TB, TB, 1)
    lmi3 = landmark_idx.astype(jnp.int32).reshape(n // TB, TB, 1)
    db2 = desc_b.reshape(1, -1)
    tb2 = tag_b.reshape(1, -1)
    ktp_w = jnp.zeros((KEY_PAD, KEY_DIM), jnp.float32).at[:key_table.shape[0]].set(key_table)
    wk = tag_W[:KEY_DIM]
    wv = tag_W[KEY_DIM:KEY_DIM + VAL_DIM]
    wd = tag_W[KEY_DIM + VAL_DIM:]
    partial = _tc_a(desc_emb, kidx3, lmi3, desc_W, db2, ktp_w, wk, wd, tb2,
                    lm_table)
    return _tc_b(partial, val_pad, wv)
